# baseline (speedup 1.0000x reference)
import functools

import jax
import jax.numpy as jnp
from jax import lax
from jax.experimental import pallas as pl
from jax.experimental.pallas import tpu as pltpu
from jax.experimental.pallas import tpu_sc as plsc

DIM = 64
NC = 2
NS = 16
NW = NC * NS

CHUNK = 128
KC = 2
SLAB = KC * CHUNK


@functools.partial(jax.jit, static_argnames=("batch_rows",))
def _sc_gather(idx3d, table2, batch_rows):
    b_per_w = batch_rows // NW
    num_slabs = b_per_w // SLAB
    total_slabs = batch_rows // SLAB
    mesh = plsc.VectorSubcoreMesh(core_axis_name="c", subcore_axis_name="s")

    @functools.partial(
        pl.kernel,
        mesh=mesh,
        out_type=jax.ShapeDtypeStruct((total_slabs, SLAB // 2, 2 * DIM),
                                      jnp.float32),
        scratch_types=[
            pltpu.VMEM((2, KC, CHUNK), jnp.int32),
            pltpu.VMEM((2, KC, CHUNK), jnp.int32),
            pltpu.VMEM((2, SLAB, 2 * DIM), jnp.float32),
            pltpu.VMEM((2, SLAB // 2, 2 * DIM), jnp.float32),
            pltpu.SemaphoreType.DMA,
            pltpu.SemaphoreType.DMA,
        ],
        compiler_params=pltpu.CompilerParams(needs_layout_passes=False),
    )
    def body(idx_hbm, table_hbm, out_hbm, idx_v, pair_v, rows_v, out_v,
             sem0, sem1):
        wid = lax.axis_index("s") * NC + lax.axis_index("c")
        slab0 = wid * num_slabs
        sems = (sem0, sem1)

        def fire(s, p):
            pltpu.sync_copy(idx_hbm.at[slab0 + s], idx_v.at[p])
            for j in range(KC):
                for o in range(CHUNK // 16):
                    sl = pl.ds(o * 16, 16)
                    raw = idx_v.at[p].at[j][sl]
                    pair_v.at[p].at[j][sl] = lax.shift_right_logical(raw, 1)
            for j in range(KC):
                pltpu.async_copy(
                    table_hbm.at[pair_v.at[p].at[j]],
                    rows_v.at[p].at[pl.ds(j * CHUNK, CHUNK)],
                    sems[p],
                )

        def select_store(s, p):
            for j in range(KC):
                pltpu.make_async_copy(
                    table_hbm.at[pl.ds(0, CHUNK)],
                    rows_v.at[p].at[pl.ds(j * CHUNK, CHUNK)],
                    sems[p],
                ).wait()
            lanes = lax.iota(jnp.int32, 16)

            def mg_body(mg, carry):
                m_vec = mg * 16 + lanes
                jrow = mg // 8
                orow = mg % 8
                raw = idx_v.at[p].at[jrow][pl.ds(orow * 16, 16)]
                half = lax.bitwise_and(raw, 1)
                dst_row = lax.shift_right_logical(m_vec, 1)
                dst_base = lax.bitwise_and(m_vec, 1) * DIM
                for c in range(DIM):
                    vals = plsc.load_gather(
                        rows_v.at[p], [m_vec, half * DIM + c])
                    plsc.store_scatter(
                        out_v.at[p], [dst_row, dst_base + c], vals)
                return carry

            lax.fori_loop(0, SLAB // 16, mg_body, 0)
            pltpu.sync_copy(out_v.at[p], out_hbm.at[slab0 + s])

        fire(0, 0)

        def step2(s2, carry):
            for p in range(2):
                s = s2 * 2 + p

                @pl.when(s + 1 < num_slabs)
                def _():
                    fire(s + 1, (p + 1) % 2)

                select_store(s, p)
            return carry

        lax.fori_loop(0, num_slabs // 2, step2, 0)

    return body(idx3d, table2)


def kernel(input, weight):
    b, h = input.shape
    batch_rows = b * h
    idx3d = input.reshape(batch_rows // SLAB, KC, CHUNK).astype(jnp.int32)
    table2 = weight.reshape(weight.shape[0] // 2, 2 * DIM)
    out = _sc_gather(idx3d, table2, batch_rows)
    return out.reshape(b, h, DIM)


# speedup vs baseline: 1.6461x; 1.6461x over previous
import functools

import jax
import jax.numpy as jnp
from jax import lax
from jax.experimental import pallas as pl
from jax.experimental.pallas import tpu as pltpu
from jax.experimental.pallas import tpu_sc as plsc

DIM = 64
NC = 2
NS = 16
NW = NC * NS

CHUNK = 128
K = 4
SLAB = K * CHUNK

NCHUNK = 4


def _make_call(rows):
    b_per_w = rows // NW
    num_slabs = b_per_w // SLAB
    mesh = plsc.VectorSubcoreMesh(core_axis_name="c", subcore_axis_name="s")

    @functools.partial(
        pl.kernel,
        mesh=mesh,
        out_type=jax.ShapeDtypeStruct((rows // SLAB, SLAB, DIM), jnp.float32),
        scratch_types=[
            pltpu.VMEM((2, K, CHUNK), jnp.int32),
            pltpu.VMEM((2, SLAB, DIM), jnp.float32),
            pltpu.SemaphoreType.DMA,
            pltpu.SemaphoreType.DMA,
        ],
        compiler_params=pltpu.CompilerParams(use_tc_tiling_on_sc=False),
    )
    def body(idx_hbm, table_hbm, out_hbm, idx_v, rows_v, sem0, sem1):
        wid = lax.axis_index("s") * NC + lax.axis_index("c")
        slab0 = wid * num_slabs
        sems = (sem0, sem1)

        def fire(s, p):
            pltpu.sync_copy(idx_hbm.at[slab0 + s], idx_v.at[p])
            for j in range(K):
                pltpu.async_copy(
                    table_hbm.at[idx_v.at[p].at[j]],
                    rows_v.at[p].at[pl.ds(j * CHUNK, CHUNK)],
                    sems[p],
                )

        def drain_store(s, p):
            for j in range(K):
                pltpu.make_async_copy(
                    table_hbm.at[pl.ds(0, CHUNK)],
                    rows_v.at[p].at[pl.ds(j * CHUNK, CHUNK)],
                    sems[p],
                ).wait()
            pltpu.sync_copy(rows_v.at[p], out_hbm.at[slab0 + s])

        fire(0, 0)

        def step2(s2, carry):
            for p in range(2):
                s = s2 * 2 + p

                @pl.when(s + 1 < num_slabs)
                def _():
                    fire(s + 1, (p + 1) % 2)

                drain_store(s, p)
            return carry

        lax.fori_loop(0, num_slabs // 2, step2, 0)

    return body


@functools.partial(jax.jit, static_argnames=("batch_rows",))
def _sc_gather(idx3d, weight, batch_rows):
    rows_per_call = batch_rows // NCHUNK
    call = _make_call(rows_per_call)
    slabs_per_call = rows_per_call // SLAB
    outs = []
    for c in range(NCHUNK):
        idx_c = lax.slice_in_dim(idx3d, c * slabs_per_call,
                                 (c + 1) * slabs_per_call, axis=0)
        outs.append(call(idx_c, weight))
    return jnp.concatenate(outs, axis=0)


def kernel(input, weight):
    b, h = input.shape
    batch_rows = b * h
    idx3d = input.reshape(batch_rows // SLAB, K, CHUNK).astype(jnp.int32)
    out = _sc_gather(idx3d, weight, batch_rows)
    return out.reshape(b, h, DIM)


# speedup vs baseline: 2.4459x; 1.4859x over previous
import functools

import jax
import jax.numpy as jnp
from jax import lax
from jax.experimental import pallas as pl
from jax.experimental.pallas import tpu as pltpu
from jax.experimental.pallas import tpu_sc as plsc

DIM = 64
HIST = 50
NC = 2
NS = 16
NW = NC * NS

NB = 16


@functools.partial(jax.jit, static_argnames=("batch",))
def _sc_gather(idx, weight, batch):
    b_per_w = batch // NW
    num_slabs = b_per_w // NB
    mesh = plsc.VectorSubcoreMesh(core_axis_name="c", subcore_axis_name="s")

    @functools.partial(
        pl.kernel,
        mesh=mesh,
        out_type=jax.ShapeDtypeStruct((batch, HIST, DIM), jnp.float32),
        scratch_types=[
            pltpu.VMEM((2, NB, HIST), jnp.int32),
            pltpu.VMEM((2, NB, HIST, DIM), jnp.float32),
            pltpu.SemaphoreType.DMA,
            pltpu.SemaphoreType.DMA,
        ],
        compiler_params=pltpu.CompilerParams(use_tc_tiling_on_sc=False),
    )
    def body(idx_hbm, table_hbm, out_hbm, idx_v, rows_v, sem0, sem1):
        wid = lax.axis_index("s") * NC + lax.axis_index("c")
        base = wid * b_per_w
        sems = (sem0, sem1)

        def fire(s, p):
            b0 = pl.multiple_of(base + s * NB, NB)
            pltpu.sync_copy(idx_hbm.at[pl.ds(b0, NB)], idx_v.at[p])
            for i in range(NB):
                pltpu.async_copy(
                    table_hbm.at[idx_v.at[p].at[i]],
                    rows_v.at[p].at[i],
                    sems[p],
                )

        def drain_store(s, p):
            for i in range(NB):
                pltpu.make_async_copy(
                    table_hbm.at[pl.ds(0, HIST)],
                    rows_v.at[p].at[i],
                    sems[p],
                ).wait()
            b0 = pl.multiple_of(base + s * NB, NB)
            pltpu.sync_copy(rows_v.at[p], out_hbm.at[pl.ds(b0, NB)])

        fire(0, 0)

        def step2(s2, carry):
            for p in range(2):
                s = s2 * 2 + p

                @pl.when(s + 1 < num_slabs)
                def _():
                    fire(s + 1, (p + 1) % 2)

                drain_store(s, p)
            return carry

        lax.fori_loop(0, num_slabs // 2, step2, 0)

    return body(idx, weight)


def kernel(input, weight):
    b, h = input.shape
    return _sc_gather(input.astype(jnp.int32), weight, b)
